# baseline (speedup 1.0000x reference)
import functools

import jax
import jax.numpy as jnp
from jax import lax
from jax.experimental import pallas as pl
from jax.experimental.pallas import tpu as pltpu
from jax.experimental.pallas import tpu_sc as plsc

D_MODEL = 768
SEQ = 512
N_CORES = 2
N_SUBCORES = 16
N_WORKERS = N_CORES * N_SUBCORES
I_PER_W = SEQ // N_WORKERS
BLK = SEQ * D_MODEL

WIN = SEQ + I_PER_W
CH = 48
N_CHUNKS = WIN // CH

STREAM_SET = tuple(range(0, I_PER_W, 2))
DIRECT_SET = tuple(range(1, I_PER_W, 2))


def _sc_copy(emb_rev_hbm, out_hbm, table_spmem, buf0, buf1,
             gsem, ssem0, ssem1, dsem):
    wid = lax.axis_index("s") * N_CORES + lax.axis_index("c")
    base_i = wid * I_PER_W
    win0 = (SEQ - I_PER_W) - base_i
    bufs = (buf0, buf1)
    ssems = (ssem0, ssem1)

    @pl.when(lax.axis_index("s") == 0)
    def _stage():
        pltpu.sync_copy(emb_rev_hbm, table_spmem)

    def gather(p):
        return pltpu.async_copy(
            emb_rev_hbm.at[pl.ds((win0 + p * CH) * D_MODEL, CH * D_MODEL)],
            bufs[p % 2],
            gsem,
        )

    g = gather(0)
    plsc.subcore_barrier()

    direct = []
    for t in DIRECT_SET:
        i = base_i + t
        direct.append(pltpu.async_copy(
            table_spmem.at[pl.ds((SEQ - i) * D_MODEL, BLK)],
            out_hbm.at[pl.ds(i * BLK, BLK)],
            dsem,
        ))

    pending_scatters = {}
    for p in range(N_CHUNKS):
        b = p % 2
        g.wait()
        handles = []
        for t in STREAM_SET:
            s0 = max(p * CH, I_PER_W - t)
            s1 = min((p + 1) * CH, WIN - t)
            if s1 <= s0:
                continue
            dst_row = s0 - (I_PER_W - t)
            handles.append(pltpu.async_copy(
                bufs[b].at[pl.ds((s0 - p * CH) * D_MODEL, (s1 - s0) * D_MODEL)],
                out_hbm.at[pl.ds((base_i + t) * BLK + dst_row * D_MODEL,
                                 (s1 - s0) * D_MODEL)],
                ssems[b],
            ))
        pending_scatters[p] = handles
        if p + 1 < N_CHUNKS:
            if p - 1 >= 0:
                for h in pending_scatters.pop(p - 1):
                    h.wait()
            g = gather(p + 1)
    for hs in pending_scatters.values():
        for h in hs:
            h.wait()
    for c in direct:
        c.wait()


def kernel(seq_len, emb):
    del seq_len
    emb_rev = emb[::-1].reshape(-1)
    mesh = plsc.VectorSubcoreMesh(core_axis_name="c", subcore_axis_name="s")
    out_flat = pl.kernel(
        _sc_copy,
        mesh=mesh,
        out_type=jax.ShapeDtypeStruct((SEQ * SEQ * D_MODEL,), jnp.float32),
        scratch_types=[
            pltpu.VMEM_SHARED((1025 * D_MODEL,), jnp.float32),
            pltpu.VMEM((CH * D_MODEL,), jnp.float32),
            pltpu.VMEM((CH * D_MODEL,), jnp.float32),
            pltpu.SemaphoreType.DMA,
            pltpu.SemaphoreType.DMA,
            pltpu.SemaphoreType.DMA,
            pltpu.SemaphoreType.DMA,
        ],
    )(emb_rev)
    return out_flat.reshape(SEQ, SEQ, D_MODEL)


# speedup vs baseline: 1.0083x; 1.0083x over previous
import functools

import jax
import jax.numpy as jnp
from jax import lax
from jax.experimental import pallas as pl
from jax.experimental.pallas import tpu as pltpu
from jax.experimental.pallas import tpu_sc as plsc

D_MODEL = 768
SEQ = 512
N_CORES = 2
N_SUBCORES = 16
N_WORKERS = N_CORES * N_SUBCORES
I_PER_W = SEQ // N_WORKERS

CH = 64
HALO = I_PER_W
N_CHUNKS = SEQ // CH


def _sc_copy(emb_rev_hbm, out_hbm, buf0, buf1, gsem, ssem0, ssem1):
    wid = lax.axis_index("s") * N_CORES + lax.axis_index("c")
    base_i = wid * I_PER_W
    win0 = (SEQ - HALO) - base_i
    bufs = (buf0, buf1)
    ssems = (ssem0, ssem1)

    def gather(c):
        return pltpu.async_copy(
            emb_rev_hbm.at[pl.ds(win0 + c * CH, CH + HALO)],
            bufs[c % 2],
            gsem,
        )

    pending = {}
    g = gather(0)
    for c in range(N_CHUNKS):
        b = c % 2
        g.wait()
        handles = []
        for t in range(I_PER_W):
            handles.append(pltpu.async_copy(
                bufs[b].at[pl.ds(HALO - t, CH)],
                out_hbm.at[base_i + t, pl.ds(c * CH, CH), :],
                ssems[b],
            ))
        pending[c] = handles
        if c + 1 < N_CHUNKS:
            if c - 1 >= 0:
                for h in pending.pop(c - 1):
                    h.wait()
            g = gather(c + 1)
    for hs in pending.values():
        for h in hs:
            h.wait()


def kernel(seq_len, emb):
    del seq_len
    emb_rev = emb[::-1]
    mesh = plsc.VectorSubcoreMesh(core_axis_name="c", subcore_axis_name="s")
    return pl.kernel(
        _sc_copy,
        mesh=mesh,
        compiler_params=pltpu.CompilerParams(use_tc_tiling_on_sc=False),
        out_type=jax.ShapeDtypeStruct((SEQ, SEQ, D_MODEL), jnp.float32),
        scratch_types=[
            pltpu.VMEM((CH + HALO, D_MODEL), jnp.float32),
            pltpu.VMEM((CH + HALO, D_MODEL), jnp.float32),
            pltpu.SemaphoreType.DMA,
            pltpu.SemaphoreType.DMA,
            pltpu.SemaphoreType.DMA,
        ],
    )(emb_rev)


# speedup vs baseline: 1.9826x; 1.9664x over previous
import functools

import jax
import jax.numpy as jnp
from jax import lax
from jax.experimental import pallas as pl
from jax.experimental.pallas import tpu as pltpu
from jax.experimental.pallas import tpu_sc as plsc

D_MODEL = 768
SEQ = 512
VOCAB = 2 * SEQ + 1
N_CORES = 2
N_SUBCORES = 16
N_WORKERS = N_CORES * N_SUBCORES
I_PER_W = SEQ // N_WORKERS
N_PAIRS = I_PER_W // 2

CH = 64
N_CHUNKS = SEQ // CH
GROWS = CH + 8
TROWS = 1032


def _sc_copy(table8_hbm, out_hbm, buf0, buf1, gsem, ssem0, ssem1):
    wid = lax.axis_index("s") * N_CORES + lax.axis_index("c")
    base_i = wid * I_PER_W
    win0 = pl.multiple_of(SEQ - base_i, 16)
    bufs = (buf0, buf1)
    ssems = (ssem0, ssem1)

    def gather(u):
        c, t = divmod(u, N_PAIRS)
        start = TROWS * t + win0 + c * CH - t - 8 + t
        return pltpu.async_copy(
            table8_hbm.at[pl.ds(pl.multiple_of(start, 8), GROWS)],
            bufs[u % 2],
            gsem,
        )

    NU = N_CHUNKS * N_PAIRS
    pending = {}
    g = gather(0)
    for u in range(NU):
        c, t = divmod(u, N_PAIRS)
        b = u % 2
        g.wait()
        handles = [
            pltpu.async_copy(
                bufs[b].at[pl.ds(8, CH)],
                out_hbm.at[base_i + t, pl.ds(c * CH, CH), :],
                ssems[b],
            ),
            pltpu.async_copy(
                bufs[b].at[pl.ds(0, CH)],
                out_hbm.at[base_i + t + 8, pl.ds(c * CH, CH), :],
                ssems[b],
            ),
        ]
        pending[u] = handles
        if u + 1 < NU:
            if u - 1 >= 0:
                for h in pending.pop(u - 1):
                    h.wait()
            g = gather(u + 1)
    for hs in pending.values():
        for h in hs:
            h.wait()


def kernel(seq_len, emb):
    del seq_len
    emb_rev = emb[::-1]
    table8 = jnp.zeros((8 * TROWS + 8, D_MODEL), jnp.float32)
    for r in range(8):
        table8 = lax.dynamic_update_slice(table8, emb_rev, (TROWS * r + r, 0))
    mesh = plsc.VectorSubcoreMesh(core_axis_name="c", subcore_axis_name="s")
    return pl.kernel(
        _sc_copy,
        mesh=mesh,
        out_type=jax.ShapeDtypeStruct((SEQ, SEQ, D_MODEL), jnp.float32),
        scratch_types=[
            pltpu.VMEM((GROWS, D_MODEL), jnp.float32),
            pltpu.VMEM((GROWS, D_MODEL), jnp.float32),
            pltpu.SemaphoreType.DMA,
            pltpu.SemaphoreType.DMA,
            pltpu.SemaphoreType.DMA,
        ],
    )(table8)
